# baseline (speedup 1.0000x reference)
import jax
import jax.numpy as jnp
from jax.experimental import pallas as pl
from jax.experimental.pallas import tpu as pltpu

_D = 64
_K = 1024
_S = 4
_BETA = 0.25
_BM = 1024
_G = 4
_NG = _K // _G


def _stage_block(r_ref, r2_ref, c2_ref, cb_ref, cbg_ref,
                 zq_ref, rn_ref, idx_ref, loss_ref):
    r = r_ref[...]
    r2 = r2_ref[...]
    c2 = c2_ref[0]
    C = cb_ref[...]
    lane_iota = jax.lax.broadcasted_iota(jnp.int32, (_BM, _K), 1)
    group_iota = jax.lax.broadcasted_iota(jnp.int32, (_BM, 3 * _NG), 1) & (_NG - 1)
    cross = jax.lax.dot_general(
        r, C, (((1,), (1,)), ((), ())),
        preferred_element_type=jnp.float32)
    d2 = (r2 - 2.0 * cross) + c2
    dmin = jnp.min(d2, axis=1, keepdims=True)
    idx = jnp.min(jnp.where(d2 == dmin, lane_iota, _K), axis=1,
                  keepdims=True)
    grp = idx >> 2
    sub = idx & 3
    oh3 = (group_iota == grp).astype(jnp.float32)
    band = jax.lax.dot_general(
        oh3, cbg_ref[...], (((1,), (0,)), ((), ())),
        preferred_element_type=jnp.float32)
    zq = jnp.zeros_like(r)
    for j in range(_G):
        zq = zq + jnp.where(sub == j, band[:, j * _D:(j + 1) * _D], 0.0)
    diff = zq - r
    loss = jnp.sum(diff * diff)
    zq_ref[...] = zq
    rn_ref[...] = r - zq
    idx_ref[...] = idx

    @pl.when(pl.program_id(0) == 0)
    def _init():
        loss_ref[0, 0] = jnp.float32(0.0)

    loss_ref[0, 0] += loss


def _stage(r, r2, c2, C, Cg):
    n = r.shape[0]
    nblk = n // _BM
    return pl.pallas_call(
        _stage_block,
        grid=(nblk,),
        in_specs=(
            pl.BlockSpec((_BM, _D), lambda i: (i, 0)),
            pl.BlockSpec((_BM, 1), lambda i: (i, 0)),
            pl.BlockSpec((1, 1, _K), lambda i: (0, 0, 0)),
            pl.BlockSpec((_K, _D), lambda i: (0, 0)),
            pl.BlockSpec((3 * _NG, _G * _D), lambda i: (0, 0)),
        ),
        out_specs=(
            pl.BlockSpec((_BM, _D), lambda i: (i, 0)),
            pl.BlockSpec((_BM, _D), lambda i: (i, 0)),
            pl.BlockSpec((_BM, 1), lambda i: (i, 0)),
            pl.BlockSpec(memory_space=pltpu.SMEM),
        ),
        out_shape=(
            jax.ShapeDtypeStruct((n, _D), jnp.float32),
            jax.ShapeDtypeStruct((n, _D), jnp.float32),
            jax.ShapeDtypeStruct((n, 1), jnp.int32),
            jax.ShapeDtypeStruct((1, 1), jnp.float32),
        ),
    )(r, r2, c2, C, Cg)


def _split_block(g_ref, out_ref):
    x = g_ref[0]
    hi = x.astype(jnp.bfloat16).astype(jnp.float32)
    e1 = x - hi
    mid = e1.astype(jnp.bfloat16).astype(jnp.float32)
    lo = e1 - mid
    out_ref[0, 0:_NG, :] = hi
    out_ref[0, _NG:2 * _NG, :] = mid
    out_ref[0, 2 * _NG:3 * _NG, :] = lo


def _split3_grouped(grouped):
    return pl.pallas_call(
        _split_block,
        grid=(_S,),
        in_specs=(pl.BlockSpec((1, _NG, _G * _D), lambda s: (s, 0, 0)),),
        out_specs=pl.BlockSpec((1, 3 * _NG, _G * _D), lambda s: (s, 0, 0)),
        out_shape=jax.ShapeDtypeStruct((_S, 3 * _NG, _G * _D), jnp.float32),
    )(grouped)


def kernel(z, codebooks):
    shape = z.shape
    zf = z.reshape(-1, _D)
    n = zf.shape[0]
    grouped = codebooks.reshape(_S, _NG, _G * _D)
    cb_grouped = _split3_grouped(grouped)

    r = zf
    recon = None
    loss = jnp.float32(0.0)
    idx_out = []
    for s in range(_S):
        C = codebooks[s]
        r2 = jnp.sum(r * r, axis=1, keepdims=True)
        c2 = jnp.sum(C * C, axis=1).reshape(1, 1, _K)
        zq, r, idx, lp = _stage(r, r2, c2, C, cb_grouped[s])
        loss = loss + _BETA * (lp[0, 0] / (n * _D))
        idx_out.append(idx.reshape(-1))
        recon = zq if recon is None else recon + zq
    z_q_out = (zf + (recon - zf)).reshape(shape)
    return (z_q_out, loss) + tuple(idx_out)


# speedup vs baseline: 1.0521x; 1.0521x over previous
import jax
import jax.numpy as jnp
from jax.experimental import pallas as pl
from jax.experimental.pallas import tpu as pltpu

_D = 64
_K = 1024
_S = 4
_BETA = 0.25
_BM = 1024
_G = 4
_NG = _K // _G


def _stage_block(r_ref, r2_ref, c2_ref, cb_ref, cbg_ref,
                 zq_ref, rn_ref, idx_ref, loss_ref):
    r = r_ref[...]
    r2 = r2_ref[...]
    c2 = c2_ref[0]
    C = cb_ref[...]
    lane_iota = jax.lax.broadcasted_iota(jnp.int32, (_BM, _K), 1)
    group_iota = jax.lax.broadcasted_iota(jnp.int32, (_BM, 3 * _NG), 1) & (_NG - 1)
    cross = jax.lax.dot_general(
        r, C, (((1,), (1,)), ((), ())),
        preferred_element_type=jnp.float32)
    d2 = (r2 - 2.0 * cross) + c2
    dmin = jnp.min(d2, axis=1, keepdims=True)
    masked_iota = jnp.where(d2 == dmin, lane_iota, _K)
    idx = jnp.min(masked_iota, axis=1, keepdims=True)
    idx_lane = jnp.min(masked_iota, axis=1)
    grp = idx >> 2
    sub = idx & 3
    oh3 = (group_iota == grp).astype(jnp.float32)
    band = jax.lax.dot_general(
        oh3, cbg_ref[...], (((1,), (0,)), ((), ())),
        preferred_element_type=jnp.float32)
    zq = jnp.zeros_like(r)
    for j in range(_G):
        zq = zq + jnp.where(sub == j, band[:, j * _D:(j + 1) * _D], 0.0)
    diff = zq - r
    loss = jnp.sum(diff * diff)
    zq_ref[...] = zq
    rn_ref[...] = r - zq
    idx_ref[0, 0, :] = idx_lane

    @pl.when(pl.program_id(0) == 0)
    def _init():
        loss_ref[0, 0] = jnp.float32(0.0)

    loss_ref[0, 0] += loss


def _stage(r, r2, c2, C, Cg):
    n = r.shape[0]
    nblk = n // _BM
    return pl.pallas_call(
        _stage_block,
        grid=(nblk,),
        in_specs=(
            pl.BlockSpec((_BM, _D), lambda i: (i, 0)),
            pl.BlockSpec((_BM, 1), lambda i: (i, 0)),
            pl.BlockSpec((1, 1, _K), lambda i: (0, 0, 0)),
            pl.BlockSpec((_K, _D), lambda i: (0, 0)),
            pl.BlockSpec((3 * _NG, _G * _D), lambda i: (0, 0)),
        ),
        out_specs=(
            pl.BlockSpec((_BM, _D), lambda i: (i, 0)),
            pl.BlockSpec((_BM, _D), lambda i: (i, 0)),
            pl.BlockSpec((1, 1, _BM), lambda i: (i, 0, 0)),
            pl.BlockSpec(memory_space=pltpu.SMEM),
        ),
        out_shape=(
            jax.ShapeDtypeStruct((n, _D), jnp.float32),
            jax.ShapeDtypeStruct((n, _D), jnp.float32),
            jax.ShapeDtypeStruct((nblk, 1, _BM), jnp.int32),
            jax.ShapeDtypeStruct((1, 1), jnp.float32),
        ),
    )(r, r2, c2, C, Cg)


def _split_block(g_ref, out_ref):
    x = g_ref[0]
    hi = x.astype(jnp.bfloat16).astype(jnp.float32)
    e1 = x - hi
    mid = e1.astype(jnp.bfloat16).astype(jnp.float32)
    lo = e1 - mid
    out_ref[0, 0:_NG, :] = hi
    out_ref[0, _NG:2 * _NG, :] = mid
    out_ref[0, 2 * _NG:3 * _NG, :] = lo


def _split3_grouped(grouped):
    return pl.pallas_call(
        _split_block,
        grid=(_S,),
        in_specs=(pl.BlockSpec((1, _NG, _G * _D), lambda s: (s, 0, 0)),),
        out_specs=pl.BlockSpec((1, 3 * _NG, _G * _D), lambda s: (s, 0, 0)),
        out_shape=jax.ShapeDtypeStruct((_S, 3 * _NG, _G * _D), jnp.float32),
    )(grouped)


def kernel(z, codebooks):
    shape = z.shape
    zf = z.reshape(-1, _D)
    n = zf.shape[0]
    grouped = codebooks.reshape(_S, _NG, _G * _D)
    cb_grouped = _split3_grouped(grouped)

    r = zf
    recon = None
    loss = jnp.float32(0.0)
    idx_out = []
    for s in range(_S):
        C = codebooks[s]
        r2 = jnp.sum(r * r, axis=1, keepdims=True)
        c2 = jnp.sum(C * C, axis=1).reshape(1, 1, _K)
        zq, r, idx, lp = _stage(r, r2, c2, C, cb_grouped[s])
        loss = loss + _BETA * (lp[0, 0] / (n * _D))
        idx_out.append(idx.reshape(-1))
        recon = zq if recon is None else recon + zq
    z_q_out = (zf + (recon - zf)).reshape(shape)
    return (z_q_out, loss) + tuple(idx_out)
